# baseline (speedup 1.0000x reference)
import jax
import jax.numpy as jnp
from jax.experimental import pallas as pl
from jax.experimental.pallas import tpu as pltpu

_K = 512


def _body(lens_ref, x_hbm, w1_ref, b1_ref, w2_ref,
          act_ref, lp_ref, en_ref, xbuf, sc_ref, carry_ref, sems):
    b = pl.program_id(0)
    K, NC = sc_ref.shape
    Hh = carry_ref.shape[1]
    len_b = lens_ref[b]
    nch = jnp.where(len_b <= 1, 0, (len_b - 1) // K + 1)

    def _copy(c, slot):
        return pltpu.make_async_copy(
            x_hbm.at[b, pl.ds(c * K, K), :], xbuf.at[slot], sems.at[slot])

    @pl.when(nch > 0)
    def _prologue():
        _copy(0, 0).start()

    def _chunk(c, carry_unused):
        slot = jax.lax.rem(c, 2)

        @pl.when(c + 1 < nch)
        def _prefetch():
            _copy(c + 1, jax.lax.rem(c + 1, 2)).start()

        _copy(c, slot).wait()
        x = xbuf[slot].astype(jnp.bfloat16)
        r = jnp.dot(x, w1_ref[...], preferred_element_type=jnp.float32)
        p_part = r[:, :Hh]
        q_part = r[:, Hh:]
        p_shift = jnp.concatenate([carry_ref[...], p_part[:-1]], axis=0)
        h = jnp.maximum(p_shift + q_part + b1_ref[...], 0.0).astype(jnp.bfloat16)
        v = jnp.dot(h, w2_ref[...], preferred_element_type=jnp.float32)
        lane = jax.lax.broadcasted_iota(jnp.int32, (K, NC), 1)
        sc_ref[...] = jnp.where(lane == c, v, sc_ref[...])
        carry_ref[...] = p_part[-1:, :]
        return carry_unused

    jax.lax.fori_loop(0, nch, _chunk, 0)

    s_all = sc_ref[...]
    g = (jax.lax.broadcasted_iota(jnp.int32, (K, NC), 0)
         + K * jax.lax.broadcasted_iota(jnp.int32, (K, NC), 1))
    valid = (g >= 1) & (g <= len_b - 1)
    s_m = jnp.where(valid, s_all, -jnp.inf)
    m = jnp.max(s_m)
    e = jnp.where(valid, jnp.exp(s_all - m), 0.0)
    l = jnp.sum(e)
    s_clean = jnp.where(valid, s_all, 0.0)
    t = jnp.sum(e * s_clean)
    cand = jnp.where(s_m == m, g, jnp.int32(2**30))
    gmin = jnp.min(cand)
    empty = len_b <= 1
    nan = jnp.float32(jnp.nan)
    logl = jnp.log(l)
    act_ref[0, 0, 0] = jnp.where(empty, 0, gmin - 1)
    lp_ref[0, 0, 0] = jnp.where(empty, nan, -logl)
    en_ref[0, 0, 0] = jnp.where(empty, nan, m + logl - t / l)


def kernel(sequence_embedding, sentence_lengths, W1, b1, W2, b2):
    B, S, D = sequence_embedding.shape
    H = W1.shape[0]
    K = _K
    NC = S // K
    w1cat = jnp.concatenate([W1[:, :D].T, W1[:, D:].T], axis=1).astype(jnp.bfloat16)
    b1r = b1.reshape(1, H)
    w2c = W2.reshape(H, 1).astype(jnp.bfloat16)

    grid_spec = pltpu.PrefetchScalarGridSpec(
        num_scalar_prefetch=1,
        grid=(B,),
        in_specs=[
            pl.BlockSpec(memory_space=pl.ANY),
            pl.BlockSpec((D, 2 * H), lambda b, lens: (0, 0)),
            pl.BlockSpec((1, H), lambda b, lens: (0, 0)),
            pl.BlockSpec((H, 1), lambda b, lens: (0, 0)),
        ],
        out_specs=[
            pl.BlockSpec((1, 1, 1), lambda b, lens: (b, 0, 0),
                         memory_space=pltpu.SMEM),
            pl.BlockSpec((1, 1, 1), lambda b, lens: (b, 0, 0),
                         memory_space=pltpu.SMEM),
            pl.BlockSpec((1, 1, 1), lambda b, lens: (b, 0, 0),
                         memory_space=pltpu.SMEM),
        ],
        scratch_shapes=[
            pltpu.VMEM((2, K, D), jnp.float32),
            pltpu.VMEM((K, NC), jnp.float32),
            pltpu.VMEM((1, H), jnp.float32),
            pltpu.SemaphoreType.DMA((2,)),
        ],
    )
    act, lp, en = pl.pallas_call(
        _body,
        grid_spec=grid_spec,
        out_shape=[
            jax.ShapeDtypeStruct((B, 1, 1), jnp.int32),
            jax.ShapeDtypeStruct((B, 1, 1), jnp.float32),
            jax.ShapeDtypeStruct((B, 1, 1), jnp.float32),
        ],
    )(sentence_lengths, sequence_embedding, w1cat, b1r, w2c)
    return act[:, 0, 0], lp[:, 0, 0], en[:, 0, 0]


# speedup vs baseline: 1.4020x; 1.4020x over previous
import jax
import jax.numpy as jnp
from jax.experimental import pallas as pl
from jax.experimental.pallas import tpu as pltpu

_K = 1024


def _body(lens_ref, x_ref, w1_ref, b1_ref, w2_ref,
          act_ref, lp_ref, en_ref, sc_ref, carry_ref):
    b = pl.program_id(0)
    c = pl.program_id(1)
    nc = pl.num_programs(1)
    K, NC = sc_ref.shape
    len_b = lens_ref[b]

    @pl.when(c * K <= len_b - 1)
    def _compute():
        x = x_ref[0]
        x_shift = jnp.concatenate([carry_ref[...], x[:-1]], axis=0)
        pairs = jnp.concatenate([x_shift, x], axis=1).astype(jnp.bfloat16)
        r = jnp.dot(pairs, w1_ref[...], preferred_element_type=jnp.float32)
        h = jnp.maximum(r + b1_ref[...], 0.0).astype(jnp.bfloat16)
        v = jnp.dot(h, w2_ref[...], preferred_element_type=jnp.float32)
        lane = jax.lax.broadcasted_iota(jnp.int32, (K, NC), 1)
        sc_ref[...] = jnp.where(lane == c, v, sc_ref[...])
        carry_ref[...] = x[-1:, :]

    @pl.when(c == nc - 1)
    def _finalize():
        s_all = sc_ref[...]
        g = (jax.lax.broadcasted_iota(jnp.int32, (K, NC), 0)
             + K * jax.lax.broadcasted_iota(jnp.int32, (K, NC), 1))
        valid = (g >= 1) & (g <= len_b - 1)
        s_m = jnp.where(valid, s_all, -jnp.inf)
        m = jnp.max(s_m)
        e = jnp.where(valid, jnp.exp(s_all - m), 0.0)
        l = jnp.sum(e)
        s_clean = jnp.where(valid, s_all, 0.0)
        t = jnp.sum(e * s_clean)
        cand = jnp.where(s_m == m, g, jnp.int32(2**30))
        gmin = jnp.min(cand)
        empty = len_b <= 1
        nan = jnp.float32(jnp.nan)
        logl = jnp.log(l)
        act_ref[0, 0, 0] = jnp.where(empty, 0, gmin - 1)
        lp_ref[0, 0, 0] = jnp.where(empty, nan, -logl)
        en_ref[0, 0, 0] = jnp.where(empty, nan, m + logl - t / l)


def kernel(sequence_embedding, sentence_lengths, W1, b1, W2, b2):
    B, S, D = sequence_embedding.shape
    H = W1.shape[0]
    K = _K
    NC = S // K
    w1t = W1.T.astype(jnp.bfloat16)
    b1r = b1.reshape(1, H)
    w2c = W2.reshape(H, 1).astype(jnp.bfloat16)

    grid_spec = pltpu.PrefetchScalarGridSpec(
        num_scalar_prefetch=1,
        grid=(B, NC),
        in_specs=[
            pl.BlockSpec(
                (1, K, D),
                lambda b, c, lens: (b, jnp.minimum(c, jnp.maximum(lens[b] - 1, 0) // K), 0)),
            pl.BlockSpec((2 * D, H), lambda b, c, lens: (0, 0)),
            pl.BlockSpec((1, H), lambda b, c, lens: (0, 0)),
            pl.BlockSpec((H, 1), lambda b, c, lens: (0, 0)),
        ],
        out_specs=[
            pl.BlockSpec((1, 1, 1), lambda b, c, lens: (b, 0, 0),
                         memory_space=pltpu.SMEM),
            pl.BlockSpec((1, 1, 1), lambda b, c, lens: (b, 0, 0),
                         memory_space=pltpu.SMEM),
            pl.BlockSpec((1, 1, 1), lambda b, c, lens: (b, 0, 0),
                         memory_space=pltpu.SMEM),
        ],
        scratch_shapes=[
            pltpu.VMEM((K, NC), jnp.float32),
            pltpu.VMEM((1, D), jnp.float32),
        ],
    )
    act, lp, en = pl.pallas_call(
        _body,
        grid_spec=grid_spec,
        out_shape=[
            jax.ShapeDtypeStruct((B, 1, 1), jnp.int32),
            jax.ShapeDtypeStruct((B, 1, 1), jnp.float32),
            jax.ShapeDtypeStruct((B, 1, 1), jnp.float32),
        ],
    )(sentence_lengths, sequence_embedding, w1t, b1r, w2c)
    return act[:, 0, 0], lp[:, 0, 0], en[:, 0, 0]


# speedup vs baseline: 1.9111x; 1.3631x over previous
import jax
import jax.numpy as jnp
from jax.experimental import pallas as pl
from jax.experimental.pallas import tpu as pltpu

_K = 512


def _body(x_ref, lens_ref, w1_ref, b1_ref, w2_ref,
          act_ref, lp_ref, en_ref, sc_ref, carry_ref):
    c = pl.program_id(0)
    nc = pl.num_programs(0)
    BK, NC = sc_ref.shape
    B, K, D = x_ref.shape

    x = x_ref[...]
    x_shift = jnp.concatenate([carry_ref[...], x[:, :-1, :]], axis=1)
    pairs = jnp.concatenate([x_shift, x], axis=2).astype(jnp.bfloat16)
    pairs2 = pairs.reshape(BK, 2 * D)
    r = jnp.dot(pairs2, w1_ref[...], preferred_element_type=jnp.float32)
    h = jnp.maximum(r + b1_ref[...], 0.0).astype(jnp.bfloat16)
    v = jnp.dot(h, w2_ref[...], preferred_element_type=jnp.float32)
    lane = jax.lax.broadcasted_iota(jnp.int32, (BK, NC), 1)
    sc_ref[...] = jnp.where(lane == c, v, sc_ref[...])
    carry_ref[...] = x[:, -1:, :]

    @pl.when(c == nc - 1)
    def _finalize():
        s3 = sc_ref[...].reshape(B, K, NC)
        pos = (jax.lax.broadcasted_iota(jnp.int32, (B, K, NC), 1)
               + K * jax.lax.broadcasted_iota(jnp.int32, (B, K, NC), 2) - 1)
        lens = lens_ref[...].reshape(B, 1, 1)
        valid = (pos >= 0) & (pos < lens - 1)
        s_m = jnp.where(valid, s3, -jnp.inf)
        m = jnp.max(jnp.max(s_m, axis=2, keepdims=True), axis=1, keepdims=True)
        e = jnp.where(valid, jnp.exp(s3 - m), 0.0)
        l = jnp.sum(jnp.sum(e, axis=2, keepdims=True), axis=1, keepdims=True)
        s_clean = jnp.where(valid, s3, 0.0)
        es = e * s_clean
        t = jnp.sum(jnp.sum(es, axis=2, keepdims=True), axis=1, keepdims=True)
        cand = jnp.where(s_m == m, pos, jnp.int32(2**30))
        pmin = jnp.min(jnp.min(cand, axis=2, keepdims=True), axis=1, keepdims=True)
        empty = lens <= 1
        nan = jnp.float32(jnp.nan)
        logl = jnp.log(l)
        act_ref[...] = jnp.where(empty, 0, pmin)
        lp_ref[...] = jnp.where(empty, nan, -logl)
        en_ref[...] = jnp.where(empty, nan, m + logl - t / l)


def kernel(sequence_embedding, sentence_lengths, W1, b1, W2, b2):
    B, S, D = sequence_embedding.shape
    H = W1.shape[0]
    K = _K
    NC = S // K
    w1t = W1.T.astype(jnp.bfloat16)
    b1r = b1.reshape(1, H)
    w2c = W2.reshape(H, 1).astype(jnp.bfloat16)
    lens2 = sentence_lengths.reshape(B, 1)

    act, lp, en = pl.pallas_call(
        _body,
        grid=(NC,),
        in_specs=[
            pl.BlockSpec((B, K, D), lambda c: (0, c, 0)),
            pl.BlockSpec((B, 1), lambda c: (0, 0)),
            pl.BlockSpec((2 * D, H), lambda c: (0, 0)),
            pl.BlockSpec((1, H), lambda c: (0, 0)),
            pl.BlockSpec((H, 1), lambda c: (0, 0)),
        ],
        out_specs=[
            pl.BlockSpec((B, 1, 1), lambda c: (0, 0, 0)),
            pl.BlockSpec((B, 1, 1), lambda c: (0, 0, 0)),
            pl.BlockSpec((B, 1, 1), lambda c: (0, 0, 0)),
        ],
        out_shape=[
            jax.ShapeDtypeStruct((B, 1, 1), jnp.int32),
            jax.ShapeDtypeStruct((B, 1, 1), jnp.float32),
            jax.ShapeDtypeStruct((B, 1, 1), jnp.float32),
        ],
        scratch_shapes=[
            pltpu.VMEM((B * K, NC), jnp.float32),
            pltpu.VMEM((B, 1, D), jnp.float32),
        ],
    )(sequence_embedding, lens2, w1t, b1r, w2c)
    return act[:, 0, 0], lp[:, 0, 0], en[:, 0, 0]


# speedup vs baseline: 2.2823x; 1.1942x over previous
import jax
import jax.numpy as jnp
from jax.experimental import pallas as pl
from jax.experimental.pallas import tpu as pltpu

_K = 512


def _body(x_ref, lenlane_ref, w1_ref, b1_ref, w2_ref,
          act_ref, lp_ref, en_ref, sc_ref, carry_ref):
    c = pl.program_id(0)
    nc = pl.num_programs(0)
    B, K, D = x_ref.shape
    L = B * nc

    @pl.when(c == 0)
    def _init():
        carry_ref[...] = jnp.zeros_like(carry_ref)

    x = x_ref[...]
    x_shift = jnp.concatenate([carry_ref[...], x[:, :-1, :]], axis=1)
    pairs = jnp.concatenate([x_shift, x], axis=2).astype(jnp.bfloat16)
    r = jnp.dot(pairs.reshape(B * K, 2 * D), w1_ref[...],
                preferred_element_type=jnp.float32)
    h = jnp.maximum(r + b1_ref[...], 0.0).astype(jnp.bfloat16)
    h3 = h.reshape(B, K, 2 * D)
    cols = [jnp.dot(h3[b], w2_ref[...], preferred_element_type=jnp.float32)
            for b in range(B)]
    vmat = jnp.concatenate(cols, axis=1)
    lane = jax.lax.broadcasted_iota(jnp.int32, (K, L), 1)
    sc_ref[...] = jnp.where(lane // B == c, jnp.tile(vmat, (1, nc)), sc_ref[...])
    carry_ref[...] = x[:, -1:, :]

    @pl.when(c == nc - 1)
    def _finalize():
        s2 = sc_ref[...]
        q = jax.lax.broadcasted_iota(jnp.int32, (K, L), 0)
        j = jax.lax.broadcasted_iota(jnp.int32, (K, L), 1)
        pos = (j // B) * K + q - 1
        lens_l = lenlane_ref[...]
        valid = (pos >= 0) & (pos < lens_l - 1)
        s_m = jnp.where(valid, s2, -jnp.inf)

        def groupred(vec, op):
            for sh in (B, 2 * B, 4 * B):
                vec = op(vec, pltpu.roll(vec, sh, 1))
            return vec

        m = groupred(jnp.max(s_m, axis=0, keepdims=True), jnp.maximum)
        e = jnp.exp(s_m - m)
        l = groupred(jnp.sum(e, axis=0, keepdims=True), jnp.add)
        t = groupred(jnp.sum(e * s2, axis=0, keepdims=True), jnp.add)
        cand = jnp.where(s_m == m, pos, jnp.int32(2**30))
        pmin = groupred(jnp.min(cand, axis=0, keepdims=True), jnp.minimum)
        le = lens_l[:, :B]
        empty = le <= 1
        nan = jnp.float32(jnp.nan)
        logl = jnp.log(l[:, :B])
        act_ref[...] = jnp.where(empty, 0, pmin[:, :B])
        lp_ref[...] = jnp.where(empty, nan, -logl)
        en_ref[...] = jnp.where(empty, nan, m[:, :B] + logl - t[:, :B] / l[:, :B])


def kernel(sequence_embedding, sentence_lengths, W1, b1, W2, b2):
    B, S, D = sequence_embedding.shape
    H = W1.shape[0]
    K = _K
    NC = S // K
    w1t = W1.T.astype(jnp.bfloat16)
    b1r = b1.reshape(1, H)
    w2c = W2.reshape(H, 1).astype(jnp.bfloat16)
    lens_lane = jnp.tile(sentence_lengths, NC).reshape(1, B * NC)

    act, lp, en = pl.pallas_call(
        _body,
        grid=(NC,),
        in_specs=[
            pl.BlockSpec((B, K, D), lambda c: (0, c, 0)),
            pl.BlockSpec((1, B * NC), lambda c: (0, 0)),
            pl.BlockSpec((2 * D, H), lambda c: (0, 0)),
            pl.BlockSpec((1, H), lambda c: (0, 0)),
            pl.BlockSpec((H, 1), lambda c: (0, 0)),
        ],
        out_specs=[
            pl.BlockSpec((1, B), lambda c: (0, 0)),
            pl.BlockSpec((1, B), lambda c: (0, 0)),
            pl.BlockSpec((1, B), lambda c: (0, 0)),
        ],
        out_shape=[
            jax.ShapeDtypeStruct((1, B), jnp.int32),
            jax.ShapeDtypeStruct((1, B), jnp.float32),
            jax.ShapeDtypeStruct((1, B), jnp.float32),
        ],
        scratch_shapes=[
            pltpu.VMEM((K, B * NC), jnp.float32),
            pltpu.VMEM((B, 1, D), jnp.float32),
        ],
    )(sequence_embedding, lens_lane, w1t, b1r, w2c)
    return act[0], lp[0], en[0]


# speedup vs baseline: 3.0091x; 1.3185x over previous
import jax
import jax.numpy as jnp
from jax.experimental import pallas as pl
from jax.experimental.pallas import tpu as pltpu

_K = 512


def _body(x_ref, lens_ref, w1_ref, b1_ref, w2_ref,
          act_ref, lp_ref, en_ref, sc_ref, carry_ref):
    c = pl.program_id(0)
    nc = pl.num_programs(0)
    B, K, D = x_ref.shape
    L = B * nc

    @pl.when(c == 0)
    def _init():
        carry_ref[...] = jnp.zeros_like(carry_ref)

    x = x_ref[...]
    x_shift = jnp.concatenate([carry_ref[...], x[:, :-1, :]], axis=1)
    pairs = jnp.concatenate([x_shift, x], axis=2).astype(jnp.bfloat16)
    w1b = w1_ref[...].astype(jnp.bfloat16)
    r = jax.lax.dot_general(pairs.reshape(B * K, 2 * D), w1b,
                            (((1,), (1,)), ((), ())),
                            preferred_element_type=jnp.float32)
    h = jnp.maximum(r + b1_ref[...], 0.0).astype(jnp.bfloat16)
    h3 = h.reshape(B, K, w1b.shape[0])
    w2rep = jnp.broadcast_to(jnp.transpose(w2_ref[...].astype(jnp.bfloat16)),
                             (w1b.shape[0], L))
    lane = jax.lax.broadcasted_iota(jnp.int32, (K, L), 1)
    acc = sc_ref[...]
    for b in range(B):
        vb = jnp.dot(h3[b], w2rep, preferred_element_type=jnp.float32)
        acc = jnp.where(lane == c * B + b, vb, acc)
    sc_ref[...] = acc
    carry_ref[...] = x[:, -1:, :]

    @pl.when(c == nc - 1)
    def _finalize():
        s2 = sc_ref[...]
        q = jax.lax.broadcasted_iota(jnp.int32, (K, L), 0)
        j = jax.lax.broadcasted_iota(jnp.int32, (K, L), 1)
        pos = (j // B) * K + q - 1
        lens_l = jnp.tile(lens_ref[...], (1, nc))
        valid = (pos >= 0) & (pos < lens_l - 1)
        s_m = jnp.where(valid, s2, -jnp.inf)

        def groupred(vec, op):
            for sh in (B, 2 * B, 4 * B):
                vec = op(vec, pltpu.roll(vec, sh, 1))
            return vec

        m = groupred(jnp.max(s_m, axis=0, keepdims=True), jnp.maximum)
        e = jnp.exp(s_m - m)
        l = groupred(jnp.sum(e, axis=0, keepdims=True), jnp.add)
        t = groupred(jnp.sum(e * s2, axis=0, keepdims=True), jnp.add)
        cand = jnp.where(s_m == m, pos, jnp.int32(2**30))
        pmin = groupred(jnp.min(cand, axis=0, keepdims=True), jnp.minimum)
        le = lens_ref[...]
        empty = le <= 1
        nan = jnp.float32(jnp.nan)
        logl = jnp.log(l[:, :B])
        act_ref[...] = jnp.where(empty, 0, pmin[:, :B])
        lp_ref[...] = jnp.where(empty, nan, -logl)
        en_ref[...] = jnp.where(empty, nan, m[:, :B] + logl - t[:, :B] / l[:, :B])


def kernel(sequence_embedding, sentence_lengths, W1, b1, W2, b2):
    B, S, D = sequence_embedding.shape
    H = W1.shape[0]
    K = _K
    NC = S // K

    act, lp, en = pl.pallas_call(
        _body,
        grid=(NC,),
        in_specs=[
            pl.BlockSpec((B, K, D), lambda c: (0, c, 0)),
            pl.BlockSpec((1, B), lambda c: (0, 0)),
            pl.BlockSpec((H, 2 * D), lambda c: (0, 0)),
            pl.BlockSpec((1, H), lambda c: (0, 0)),
            pl.BlockSpec((1, H), lambda c: (0, 0)),
        ],
        out_specs=[
            pl.BlockSpec((1, B), lambda c: (0, 0)),
            pl.BlockSpec((1, B), lambda c: (0, 0)),
            pl.BlockSpec((1, B), lambda c: (0, 0)),
        ],
        out_shape=[
            jax.ShapeDtypeStruct((1, B), jnp.int32),
            jax.ShapeDtypeStruct((1, B), jnp.float32),
            jax.ShapeDtypeStruct((1, B), jnp.float32),
        ],
        scratch_shapes=[
            pltpu.VMEM((K, B * NC), jnp.float32),
            pltpu.VMEM((B, 1, D), jnp.float32),
        ],
    )(sequence_embedding, sentence_lengths.reshape(1, B), W1, b1.reshape(1, H),
      W2)
    return act[0], lp[0], en[0]
